# baseline (speedup 1.0000x reference)
import functools

import jax
import jax.numpy as jnp
from jax import lax
from jax.experimental import pallas as pl
from jax.experimental.pallas import tpu as pltpu
from jax.experimental.pallas import tpu_sc as plsc

_N_TYPES = (20000, 15000, 15000)
_ETYPES = ((0, 1), (2, 1), (2, 0), (0, 0), (1, 2), (1, 0), (0, 0), (1, 1), (2, 2))
_E = 90000
_D_IN, _D_HID, _D_OUT, _N_LAYERS = 128, 64, 8, 3

_NC, _NS = 2, 16
_CH = 128
_EPAD = 90112
_NROW = _EPAD // _CH
_RPT = _NROW // _NS

_SRC_GROUPS = ((0, 3, 6), (4, 5, 7), (1, 2, 8))
_SLAB = {e: k for grp in _SRC_GROUPS for k, e in enumerate(grp)}
_DST_GROUPS = ((2, 3, 5, 6), (0, 1, 7), (4, 8))

_JOBS = (
    ((0, 0, 44), (2, 0, 44), (4, 0, 44), (6, 0, 44), (8, 0, 24)),
    ((1, 0, 44), (3, 0, 44), (5, 0, 44), (7, 0, 44), (8, 24, 20)),
)
_JOB_OUT = ((0, 1, 2, 3, 4), (5, 6, 7, 8, 9))
_DST_CONTRIBS = (
    ((1, 2), (3, 6), (6, 3), (7, 5)),
    ((0, 0), (5, 1), (8, 7)),
    ((2, 4), (4, 8), (9, 8)),
)

_ZCH = 64


def _agg_rows(n):
    q = -(-(n + 1) // _NS)
    q = -(-q // _ZCH) * _ZCH
    return _NS * q


def _deg_rows(n):
    q = -(-(n + 1) // _NS)
    q = -(-q // 128) * 128
    return _NS * q


_RCH = 2

_AGG_SIZES = tuple(_agg_rows(_N_TYPES[_ETYPES[e][1]])
                   for core in range(_NC) for e, _, _ in _JOBS[core])
_AGG_OFF = tuple(sum(_AGG_SIZES[:j]) for j in range(len(_AGG_SIZES)))
_AGG_TOTAL = sum(_AGG_SIZES)
_DEG_SIZES = tuple(_deg_rows(_N_TYPES[_ETYPES[e][side]])
                   for side in range(2) for e in range(9))
_DEG_OFF = tuple(sum(_DEG_SIZES[:j]) for j in range(len(_DEG_SIZES)))
_DEG_TOTAL = sum(_DEG_SIZES)


def _feat_scatter_body(*all_args):
    ztabs_flat = all_args[:9]
    gidx, sdst, zrows = all_args[9:12]
    rest = all_args[12:]
    out = rest[0]
    gidx_v, sdst_v, rows0_v, rows1_v, zeros_v, spmem, sem0, sem1 = rest[1:]
    ztabs = (ztabs_flat[0:3], ztabs_flat[3:6], ztabs_flat[6:9])
    rows_v = (rows0_v, rows1_v)
    sems = (sem0, sem1)
    cid = lax.axis_index("c")
    sid = lax.axis_index("s")
    pltpu.sync_copy(zrows, zeros_v)

    for core in range(_NC):
        for ji, (e, row0, rpt) in enumerate(_JOBS[core]):
            n_d = _N_TYPES[_ETYPES[e][1]]
            rows = _agg_rows(n_d)
            q = rows // _NS
            aoff = _AGG_OFF[_JOB_OUT[core][ji]]

            @pl.when(cid == core)
            def _(e=e, row0=row0, rpt=rpt, q=q, aoff=aoff):
                for i in range(q // _ZCH):
                    pltpu.sync_copy(
                        zeros_v, spmem.at[pl.ds(sid * q + i * _ZCH, _ZCH)])
                plsc.subcore_barrier()
                ne = rpt * _CH
                pltpu.sync_copy(gidx.at[e, sid, pl.ds(row0 * _CH, ne)],
                                gidx_v.at[pl.ds(0, ne)])
                pltpu.sync_copy(sdst.at[e, sid, pl.ds(row0 * _CH, ne)],
                                sdst_v.at[pl.ds(0, ne)])
                ztab = ztabs[_ETYPES[e][0]][_SLAB[e]]

                ec = _RCH * _CH
                nch = rpt // _RCH
                descs = [None, None]
                descs[0] = pltpu.async_copy(
                    ztab.at[gidx_v.at[pl.ds(0, ec)]], rows_v[0], sems[0])
                for ci in range(nch):
                    b = ci % 2
                    descs[b].wait()
                    if ci + 1 < nch:
                        nb = (ci + 1) % 2
                        descs[nb] = pltpu.async_copy(
                            ztab.at[gidx_v.at[pl.ds((ci + 1) * ec, ec)]],
                            rows_v[nb], sems[nb])
                    pltpu.sync_copy(
                        rows_v[b],
                        spmem.at[sdst_v.at[pl.ds(ci * ec, ec)]], add=True)

                plsc.subcore_barrier()
                pltpu.sync_copy(spmem.at[pl.ds(sid * q, q)],
                                out.at[pl.ds(aoff + sid * q, q)])
                plsc.subcore_barrier()


def _sc_feat_scatter(ztabs_flat, gidx, sdst, zrows):
    out_type = jax.ShapeDtypeStruct((_AGG_TOTAL, _D_HID), jnp.float32)
    mesh = plsc.VectorSubcoreMesh(
        core_axis_name="c", subcore_axis_name="s", num_cores=_NC,
        num_subcores=_NS)
    max_rows = max(_agg_rows(n) for n in _N_TYPES)
    call = pl.kernel(
        _feat_scatter_body,
        out_type=out_type,
        mesh=mesh,
        scratch_types=[
            pltpu.VMEM((_RPT * _CH,), jnp.int32),
            pltpu.VMEM((_RPT * _CH,), jnp.int32),
            pltpu.VMEM((_RCH * _CH, _D_HID), jnp.float32),
            pltpu.VMEM((_RCH * _CH, _D_HID), jnp.float32),
            pltpu.VMEM((_ZCH, _D_HID), jnp.float32),
            pltpu.VMEM_SHARED((max_rows, _D_HID), jnp.float32),
            pltpu.SemaphoreType.DMA,
            pltpu.SemaphoreType.DMA,
        ],
        compiler_params=pltpu.CompilerParams(use_tc_tiling_on_sc=False),
    )
    return call(*ztabs_flat, gidx, sdst, zrows)


def _deg_body(degsrc, degdst, ones, zrows8, *rest):
    out = rest[0]
    idx_v, ones_v, zeros_v, spmem = rest[1:]
    cid = lax.axis_index("c")
    sid = lax.axis_index("s")
    pltpu.sync_copy(ones, ones_v)
    pltpu.sync_copy(zrows8, zeros_v)

    for core in range(_NC):
        idx_hbm = degsrc if core == 0 else degdst

        @pl.when(cid == core)
        def _(core=core, idx_hbm=idx_hbm):
            for e in range(9):
                s, d = _ETYPES[e]
                n = _N_TYPES[s] if core == 0 else _N_TYPES[d]
                rows = _deg_rows(n)
                q = rows // _NS
                doff = _DEG_OFF[core * 9 + e]
                pltpu.sync_copy(zeros_v.at[pl.ds(0, q)],
                                spmem.at[pl.ds(sid * q, q)])
                plsc.subcore_barrier()
                pltpu.sync_copy(idx_hbm.at[e, sid], idx_v)
                ec = _RCH * _CH

                def chunk(j, carry):
                    pltpu.sync_copy(
                        ones_v, spmem.at[idx_v.at[pl.ds(j * ec, ec)]],
                        add=True)
                    return carry

                lax.fori_loop(0, _RPT // _RCH, chunk, 0, unroll=False)
                plsc.subcore_barrier()
                pltpu.sync_copy(spmem.at[pl.ds(sid * q, q)],
                                out.at[pl.ds(doff + sid * q, q)])
                plsc.subcore_barrier()


def _sc_degrees(degsrc, degdst, ones, zrows8):
    out_type = jax.ShapeDtypeStruct((_DEG_TOTAL, 8), jnp.float32)
    mesh = plsc.VectorSubcoreMesh(
        core_axis_name="c", subcore_axis_name="s", num_cores=_NC,
        num_subcores=_NS)
    max_q = max(_deg_rows(n) for n in _N_TYPES) // _NS
    call = pl.kernel(
        _deg_body,
        out_type=out_type,
        mesh=mesh,
        scratch_types=[
            pltpu.VMEM((_RPT * _CH,), jnp.int32),
            pltpu.VMEM((_RCH * _CH, 8), jnp.float32),
            pltpu.VMEM((max_q, 8), jnp.float32),
            pltpu.VMEM_SHARED((max(_deg_rows(n) for n in _N_TYPES), 8),
                              jnp.float32),
        ],
        compiler_params=pltpu.CompilerParams(use_tc_tiling_on_sc=False),
    )
    return call(degsrc, degdst, ones, zrows8)


_B = 1024


def _b0(*block):
    return pl.BlockSpec(block, lambda i: (0,) * len(block))


def _brow(r, *block):
    return pl.BlockSpec(block, lambda i, r=r: (r,) + (0,) * (len(block) - 1))


def _a_tail(h, w_refs, ocnt_refs, out_refs):
    for k in range(3):
        t = jnp.dot(h, w_refs[k][0, 0],
                    preferred_element_type=jnp.float32)
        s = lax.rsqrt(jnp.maximum(ocnt_refs[k][...][:, :1], 1.0))
        out_refs[k][...] = t * s


def _a_specs_args(l, s, Wc, deg):
    in_specs, args = [], []
    for e in _SRC_GROUPS[s]:
        in_specs.append(pl.BlockSpec(
            (1, 1, _D_HID, _D_HID), lambda i, l=l, e=e: (l, e, 0, 0)))
        args.append(Wc)
    for e in _SRC_GROUPS[s]:
        ob = _DEG_OFF[e] // _B
        in_specs.append(pl.BlockSpec((_B, 8), lambda i, ob=ob: (ob + i, 0)))
        args.append(deg)
    return in_specs, args


def _embed_a_body(s, x_ref, wemb_ref, bemb_ref, w0, w1, w2, c0, c1, c2,
                  o0, o1, o2):
    h = jnp.dot(x_ref[...], wemb_ref[0],
                preferred_element_type=jnp.float32)
    h = h + bemb_ref[s:s + 1, :]
    _a_tail(h, (w0, w1, w2), (c0, c1, c2), (o0, o1, o2))


def _z_out(n):
    return (
        [pl.BlockSpec((_B, _D_HID), lambda i: (i, 0))] * 3,
        [jax.ShapeDtypeStruct((n, _D_HID), jnp.float32)] * 3,
    )


def _embed_a(x, s, W_emb, b_emb, Wc, deg):
    n = x.shape[0]
    in_specs = [pl.BlockSpec((_B, _D_IN), lambda i: (i, 0)),
                _brow(s, 1, _D_IN, _D_HID),
                _b0(3, _D_HID)]
    a_specs, a_args = _a_specs_args(0, s, Wc, deg)
    in_specs += a_specs
    out_specs, out_shape = _z_out(n)
    return pl.pallas_call(
        functools.partial(_embed_a_body, s),
        grid=(-(-n // _B),),
        in_specs=in_specs,
        out_specs=out_specs,
        out_shape=out_shape,
    )(x, W_emb, b_emb, *a_args)


def _c_mid(ncontrib, bias_rows, d, refs):
    agg_refs = refs[:ncontrib]
    cnt_refs = refs[ncontrib:2 * ncontrib]
    k = 2 * ncontrib
    bc_ref, g_ref, b_ref = refs[k], refs[k + 1], refs[k + 2]
    acc = jnp.zeros((_B, _D_HID), jnp.float32)
    for i in range(ncontrib):
        s = lax.rsqrt(jnp.maximum(cnt_refs[i][...][:, :1], 1.0))
        acc = acc + agg_refs[i][...] * s
    bias = jnp.zeros((1, _D_HID), jnp.float32)
    for r in bias_rows:
        bias = bias + bc_ref[0, r:r + 1, :]
    acc = acc + bias
    mu = jnp.mean(acc, axis=-1, keepdims=True)
    var = jnp.mean((acc - mu) ** 2, axis=-1, keepdims=True)
    y = (acc - mu) * lax.rsqrt(var + 1e-5) * g_ref[d:d + 1, :] \
        + b_ref[d:d + 1, :]
    return jnp.maximum(y, 0.0), 2 * ncontrib + 3


def _fused_ca_body(ncontrib, bias_rows, d, *refs):
    y, k = _c_mid(ncontrib, bias_rows, d, refs)
    w_refs = refs[k:k + 3]
    ocnt_refs = refs[k + 3:k + 6]
    out_refs = refs[k + 6:k + 9]
    _a_tail(y, w_refs, ocnt_refs, out_refs)


def _final_c_body(ncontrib, bias_rows, d, *refs):
    y, k = _c_mid(ncontrib, bias_rows, d, refs)
    wcls_ref, bcls_ref, out_ref = refs[k], refs[k + 1], refs[k + 2]
    y = jnp.dot(y, wcls_ref[0], preferred_element_type=jnp.float32)
    out_ref[...] = y + bcls_ref[d:d + 1, :]


def _c_specs_args(l, jobs, etypes, agg, deg, bc, ln_g, ln_b):
    in_specs = []
    args = []
    for j in jobs:
        ob = _AGG_OFF[j] // _B
        in_specs.append(pl.BlockSpec((_B, _D_HID),
                                     lambda i, ob=ob: (ob + i, 0)))
        args.append(agg)
    for e in etypes:
        ob = _DEG_OFF[9 + e] // _B
        in_specs.append(pl.BlockSpec((_B, 8), lambda i, ob=ob: (ob + i, 0)))
        args.append(deg)
    in_specs += [_brow(l, 1, 9, _D_HID), _b0(3, _D_HID), _b0(3, _D_HID)]
    args += [bc, ln_g, ln_b]
    return in_specs, args


def _fused_ca(n, l, d, jobs, etypes, agg, deg, bc, bias_rows, ln_g, ln_b,
              Wc):
    in_specs, args = _c_specs_args(l, jobs, etypes, agg, deg, bc, ln_g, ln_b)
    a_specs, a_args = _a_specs_args(l + 1, d, Wc, deg)
    in_specs += a_specs
    args += a_args
    out_specs, out_shape = _z_out(n)
    return pl.pallas_call(
        functools.partial(_fused_ca_body, len(jobs), tuple(bias_rows), d),
        grid=(-(-n // _B),),
        in_specs=in_specs,
        out_specs=out_specs,
        out_shape=out_shape,
    )(*args)


def _final_c(n, l, d, jobs, etypes, agg, deg, bc, bias_rows, ln_g, ln_b,
             Wcls, bcls):
    in_specs, args = _c_specs_args(l, jobs, etypes, agg, deg, bc, ln_g, ln_b)
    in_specs += [_brow(d, 1, _D_HID, _D_OUT), _b0(3, _D_OUT)]
    args += [Wcls, bcls]
    return pl.pallas_call(
        functools.partial(_final_c_body, len(jobs), tuple(bias_rows), d),
        grid=(-(-n // _B),),
        in_specs=in_specs,
        out_specs=pl.BlockSpec((_B, _D_OUT), lambda i: (i, 0)),
        out_shape=jax.ShapeDtypeStruct((n, _D_OUT), jnp.float32),
    )(*args)


def kernel(assmpt_feat, rule_feat, non_assmpt_feat, W_emb, b_emb, Wc, bc,
           ln_g, ln_b, Wcls, bcls, edges_src, edges_dst):
    feats = (assmpt_feat, rule_feat, non_assmpt_feat)
    npad = _EPAD - _E

    gidx_l, degsrc_l, sdst_l = [], [], []
    for e, (s, d) in enumerate(_ETYPES):
        n_s, n_d = _N_TYPES[s], _N_TYPES[d]
        src_e, dst_e = edges_src[e], edges_dst[e]
        gidx_l.append(jnp.concatenate(
            [src_e, jnp.zeros((npad,), jnp.int32)]))
        degsrc_l.append(jnp.concatenate(
            [src_e, jnp.full((npad,), n_s, jnp.int32)]))
        sdst_l.append(jnp.concatenate(
            [dst_e, jnp.full((npad,), n_d, jnp.int32)]))
    gidx = jnp.stack(gidx_l).reshape(9, _NS, _RPT * _CH)
    degsrc = jnp.stack(degsrc_l).reshape(9, _NS, _RPT * _CH)
    sdst = jnp.stack(sdst_l).reshape(9, _NS, _RPT * _CH)
    ones8 = jnp.ones((_RCH * _CH, 8), jnp.float32)
    max_q8 = max(_deg_rows(n) for n in _N_TYPES) // _NS
    zrows8 = jnp.zeros((max_q8, 8), jnp.float32)
    zrows = jnp.zeros((_ZCH, _D_HID), jnp.float32)

    deg = _sc_degrees(degsrc, sdst, ones8, zrows8)

    ztabs = [_embed_a(feats[s], s, W_emb, b_emb, Wc, deg)
             for s in range(3)]

    out = [None, None, None]
    for l in range(_N_LAYERS):
        agg = _sc_feat_scatter(
            [t for zs in ztabs for t in zs], gidx, sdst, zrows)
        last = l == _N_LAYERS - 1
        nxt = []
        for d in range(3):
            n_d = _N_TYPES[d]
            contribs = _DST_CONTRIBS[d]
            jobs = [j for j, _ in contribs]
            etypes = [e for _, e in contribs]
            if last:
                out[d] = _final_c(
                    n_d, l, d, jobs, etypes, agg, deg, bc, _DST_GROUPS[d],
                    ln_g, ln_b, Wcls, bcls)
            else:
                nxt.append(_fused_ca(
                    n_d, l, d, jobs, etypes, agg, deg, bc, _DST_GROUPS[d],
                    ln_g, ln_b, Wc))
        ztabs = nxt

    return out[0], out[1], out[2]


# speedup vs baseline: 1.1344x; 1.1344x over previous
import functools

import jax
import jax.numpy as jnp
from jax import lax
from jax.experimental import pallas as pl
from jax.experimental.pallas import tpu as pltpu
from jax.experimental.pallas import tpu_sc as plsc

_N_TYPES = (20000, 15000, 15000)
_ETYPES = ((0, 1), (2, 1), (2, 0), (0, 0), (1, 2), (1, 0), (0, 0), (1, 1), (2, 2))
_E = 90000
_D_IN, _D_HID, _D_OUT, _N_LAYERS = 128, 64, 8, 3

_NC, _NS = 2, 16
_CH = 128
_EPAD = 90112
_NROW = _EPAD // _CH
_RPT = _NROW // _NS

_SRC_GROUPS = ((0, 3, 6), (4, 5, 7), (1, 2, 8))
_SLAB = {e: k for grp in _SRC_GROUPS for k, e in enumerate(grp)}
_DST_GROUPS = ((2, 3, 5, 6), (0, 1, 7), (4, 8))

def _sc_jobs(s):
    a, b, c = _SRC_GROUPS[s]
    return (((a, 0, 44), (c, 0, 24)), ((b, 0, 44), (c, 24, 20)))


def _sc_region_etypes(s):
    a, b, c = _SRC_GROUPS[s]
    return (a, c, b, c)


_ZCH = 64


def _agg_rows(n):
    q = -(-(n + 1) // _NS)
    q = -(-q // _ZCH) * _ZCH
    return _NS * q


def _deg_rows(n):
    q = -(-(n + 1) // _NS)
    q = -(-q // 128) * 128
    return _NS * q


_RCH = 2

def _region_layout(s):
    sizes = tuple(_agg_rows(_N_TYPES[_ETYPES[e][1]])
                  for e in _sc_region_etypes(s))
    offs = tuple(sum(sizes[:j]) for j in range(len(sizes)))
    return offs, sum(sizes)


_DST_CONTRIBS = tuple(
    tuple((s, r, e) for s in range(3)
          for r, e in enumerate(_sc_region_etypes(s))
          if _ETYPES[e][1] == d)
    for d in range(3))

_DEG_SIZES = tuple(_deg_rows(_N_TYPES[_ETYPES[e][side]])
                   for side in range(2) for e in range(9))
_DEG_OFF = tuple(sum(_DEG_SIZES[:j]) for j in range(len(_DEG_SIZES)))
_DEG_TOTAL = sum(_DEG_SIZES)


def _feat_scatter_body(s, z0, z1, z2, gidx, sdst, zrows, *rest):
    out = rest[0]
    gidx_v, sdst_v, rows0_v, rows1_v, zeros_v, spmem, sem0, sem1 = rest[1:]
    ztabs = (z0, z1, z2)
    rows_v = (rows0_v, rows1_v)
    sems = (sem0, sem1)
    cid = lax.axis_index("c")
    sid = lax.axis_index("s")
    pltpu.sync_copy(zrows, zeros_v)
    offs, _ = _region_layout(s)
    jobs = _sc_jobs(s)

    for core in range(_NC):
        for ji, (e, row0, rpt) in enumerate(jobs[core]):
            n_d = _N_TYPES[_ETYPES[e][1]]
            rows = _agg_rows(n_d)
            q = rows // _NS
            aoff = offs[core * 2 + ji]

            @pl.when(cid == core)
            def _(e=e, row0=row0, rpt=rpt, q=q, aoff=aoff):
                for i in range(q // _ZCH):
                    pltpu.sync_copy(
                        zeros_v, spmem.at[pl.ds(sid * q + i * _ZCH, _ZCH)])
                plsc.subcore_barrier()
                ne = rpt * _CH
                pltpu.sync_copy(gidx.at[e, sid, pl.ds(row0 * _CH, ne)],
                                gidx_v.at[pl.ds(0, ne)])
                pltpu.sync_copy(sdst.at[e, sid, pl.ds(row0 * _CH, ne)],
                                sdst_v.at[pl.ds(0, ne)])
                ztab = ztabs[_SLAB[e]]

                ec = _RCH * _CH
                nch = rpt // _RCH
                descs = [None, None]
                descs[0] = pltpu.async_copy(
                    ztab.at[gidx_v.at[pl.ds(0, ec)]], rows_v[0], sems[0])
                for ci in range(nch):
                    b = ci % 2
                    descs[b].wait()
                    if ci + 1 < nch:
                        nb = (ci + 1) % 2
                        descs[nb] = pltpu.async_copy(
                            ztab.at[gidx_v.at[pl.ds((ci + 1) * ec, ec)]],
                            rows_v[nb], sems[nb])
                    pltpu.sync_copy(
                        rows_v[b],
                        spmem.at[sdst_v.at[pl.ds(ci * ec, ec)]], add=True)

                plsc.subcore_barrier()
                pltpu.sync_copy(spmem.at[pl.ds(sid * q, q)],
                                out.at[pl.ds(aoff + sid * q, q)])
                plsc.subcore_barrier()


def _sc_feat_scatter(s, ztabs3, gidx, sdst, zrows):
    _, total = _region_layout(s)
    out_type = jax.ShapeDtypeStruct((total, _D_HID), jnp.float32)
    mesh = plsc.VectorSubcoreMesh(
        core_axis_name="c", subcore_axis_name="s", num_cores=_NC,
        num_subcores=_NS)
    max_rows = max(_agg_rows(n) for n in _N_TYPES)
    call = pl.kernel(
        functools.partial(_feat_scatter_body, s),
        out_type=out_type,
        mesh=mesh,
        scratch_types=[
            pltpu.VMEM((_RPT * _CH,), jnp.int32),
            pltpu.VMEM((_RPT * _CH,), jnp.int32),
            pltpu.VMEM((_RCH * _CH, _D_HID), jnp.float32),
            pltpu.VMEM((_RCH * _CH, _D_HID), jnp.float32),
            pltpu.VMEM((_ZCH, _D_HID), jnp.float32),
            pltpu.VMEM_SHARED((max_rows, _D_HID), jnp.float32),
            pltpu.SemaphoreType.DMA,
            pltpu.SemaphoreType.DMA,
        ],
        compiler_params=pltpu.CompilerParams(use_tc_tiling_on_sc=False),
    )
    return call(*ztabs3, gidx, sdst, zrows)


def _deg_body(degsrc, degdst, ones, zrows8, *rest):
    out = rest[0]
    idx_v, ones_v, zeros_v, spmem = rest[1:]
    cid = lax.axis_index("c")
    sid = lax.axis_index("s")
    pltpu.sync_copy(ones, ones_v)
    pltpu.sync_copy(zrows8, zeros_v)

    for core in range(_NC):
        idx_hbm = degsrc if core == 0 else degdst

        @pl.when(cid == core)
        def _(core=core, idx_hbm=idx_hbm):
            for e in range(9):
                s, d = _ETYPES[e]
                n = _N_TYPES[s] if core == 0 else _N_TYPES[d]
                rows = _deg_rows(n)
                q = rows // _NS
                doff = _DEG_OFF[core * 9 + e]
                pltpu.sync_copy(zeros_v.at[pl.ds(0, q)],
                                spmem.at[pl.ds(sid * q, q)])
                plsc.subcore_barrier()
                pltpu.sync_copy(idx_hbm.at[e, sid], idx_v)
                ec = _RCH * _CH

                def chunk(j, carry):
                    pltpu.sync_copy(
                        ones_v, spmem.at[idx_v.at[pl.ds(j * ec, ec)]],
                        add=True)
                    return carry

                lax.fori_loop(0, _RPT // _RCH, chunk, 0, unroll=False)
                plsc.subcore_barrier()
                pltpu.sync_copy(spmem.at[pl.ds(sid * q, q)],
                                out.at[pl.ds(doff + sid * q, q)])
                plsc.subcore_barrier()


def _sc_degrees(degsrc, degdst, ones, zrows8):
    out_type = jax.ShapeDtypeStruct((_DEG_TOTAL, 8), jnp.float32)
    mesh = plsc.VectorSubcoreMesh(
        core_axis_name="c", subcore_axis_name="s", num_cores=_NC,
        num_subcores=_NS)
    max_q = max(_deg_rows(n) for n in _N_TYPES) // _NS
    call = pl.kernel(
        _deg_body,
        out_type=out_type,
        mesh=mesh,
        scratch_types=[
            pltpu.VMEM((_RPT * _CH,), jnp.int32),
            pltpu.VMEM((_RCH * _CH, 8), jnp.float32),
            pltpu.VMEM((max_q, 8), jnp.float32),
            pltpu.VMEM_SHARED((max(_deg_rows(n) for n in _N_TYPES), 8),
                              jnp.float32),
        ],
        compiler_params=pltpu.CompilerParams(use_tc_tiling_on_sc=False),
    )
    return call(degsrc, degdst, ones, zrows8)


_B = 1024


def _b0(*block):
    return pl.BlockSpec(block, lambda i: (0,) * len(block))


def _brow(r, *block):
    return pl.BlockSpec(block, lambda i, r=r: (r,) + (0,) * (len(block) - 1))


def _a_tail(h, w_refs, ocnt_refs, out_refs):
    for k in range(3):
        t = jnp.dot(h, w_refs[k][0, 0],
                    preferred_element_type=jnp.float32)
        s = lax.rsqrt(jnp.maximum(ocnt_refs[k][...][:, :1], 1.0))
        out_refs[k][...] = t * s


def _a_specs_args(l, s, Wc, deg):
    in_specs, args = [], []
    for e in _SRC_GROUPS[s]:
        in_specs.append(pl.BlockSpec(
            (1, 1, _D_HID, _D_HID), lambda i, l=l, e=e: (l, e, 0, 0)))
        args.append(Wc)
    for e in _SRC_GROUPS[s]:
        ob = _DEG_OFF[e] // _B
        in_specs.append(pl.BlockSpec((_B, 8), lambda i, ob=ob: (ob + i, 0)))
        args.append(deg)
    return in_specs, args


def _embed_a_body(s, x_ref, wemb_ref, bemb_ref, w0, w1, w2, c0, c1, c2,
                  o0, o1, o2):
    h = jnp.dot(x_ref[...], wemb_ref[0],
                preferred_element_type=jnp.float32)
    h = h + bemb_ref[s:s + 1, :]
    _a_tail(h, (w0, w1, w2), (c0, c1, c2), (o0, o1, o2))


def _z_out(n):
    return (
        [pl.BlockSpec((_B, _D_HID), lambda i: (i, 0))] * 3,
        [jax.ShapeDtypeStruct((n, _D_HID), jnp.float32)] * 3,
    )


def _embed_a(x, s, W_emb, b_emb, Wc, deg):
    n = x.shape[0]
    in_specs = [pl.BlockSpec((_B, _D_IN), lambda i: (i, 0)),
                _brow(s, 1, _D_IN, _D_HID),
                _b0(3, _D_HID)]
    a_specs, a_args = _a_specs_args(0, s, Wc, deg)
    in_specs += a_specs
    out_specs, out_shape = _z_out(n)
    return pl.pallas_call(
        functools.partial(_embed_a_body, s),
        grid=(-(-n // _B),),
        in_specs=in_specs,
        out_specs=out_specs,
        out_shape=out_shape,
    )(x, W_emb, b_emb, *a_args)


def _c_mid(ncontrib, bias_rows, d, refs):
    agg_refs = refs[:ncontrib]
    cnt_refs = refs[ncontrib:2 * ncontrib]
    k = 2 * ncontrib
    bc_ref, g_ref, b_ref = refs[k], refs[k + 1], refs[k + 2]
    acc = jnp.zeros((_B, _D_HID), jnp.float32)
    for i in range(ncontrib):
        s = lax.rsqrt(jnp.maximum(cnt_refs[i][...][:, :1], 1.0))
        acc = acc + agg_refs[i][...] * s
    bias = jnp.zeros((1, _D_HID), jnp.float32)
    for r in bias_rows:
        bias = bias + bc_ref[0, r:r + 1, :]
    acc = acc + bias
    mu = jnp.mean(acc, axis=-1, keepdims=True)
    var = jnp.mean((acc - mu) ** 2, axis=-1, keepdims=True)
    y = (acc - mu) * lax.rsqrt(var + 1e-5) * g_ref[d:d + 1, :] \
        + b_ref[d:d + 1, :]
    return jnp.maximum(y, 0.0), 2 * ncontrib + 3


def _fused_ca_body(ncontrib, bias_rows, d, *refs):
    y, k = _c_mid(ncontrib, bias_rows, d, refs)
    w_refs = refs[k:k + 3]
    ocnt_refs = refs[k + 3:k + 6]
    out_refs = refs[k + 6:k + 9]
    _a_tail(y, w_refs, ocnt_refs, out_refs)


def _final_c_body(ncontrib, bias_rows, d, *refs):
    y, k = _c_mid(ncontrib, bias_rows, d, refs)
    wcls_ref, bcls_ref, out_ref = refs[k], refs[k + 1], refs[k + 2]
    y = jnp.dot(y, wcls_ref[0], preferred_element_type=jnp.float32)
    out_ref[...] = y + bcls_ref[d:d + 1, :]


def _c_specs_args(l, d, aggs3, deg, bc, ln_g, ln_b):
    in_specs = []
    args = []
    for s, r, e in _DST_CONTRIBS[d]:
        ob = _region_layout(s)[0][r] // _B
        in_specs.append(pl.BlockSpec((_B, _D_HID),
                                     lambda i, ob=ob: (ob + i, 0)))
        args.append(aggs3[s])
    for s, r, e in _DST_CONTRIBS[d]:
        ob = _DEG_OFF[9 + e] // _B
        in_specs.append(pl.BlockSpec((_B, 8), lambda i, ob=ob: (ob + i, 0)))
        args.append(deg)
    in_specs += [_brow(l, 1, 9, _D_HID), _b0(3, _D_HID), _b0(3, _D_HID)]
    args += [bc, ln_g, ln_b]
    return in_specs, args


def _fused_ca(n, l, d, aggs3, deg, bc, bias_rows, ln_g, ln_b, Wc):
    in_specs, args = _c_specs_args(l, d, aggs3, deg, bc, ln_g, ln_b)
    a_specs, a_args = _a_specs_args(l + 1, d, Wc, deg)
    in_specs += a_specs
    args += a_args
    out_specs, out_shape = _z_out(n)
    return pl.pallas_call(
        functools.partial(_fused_ca_body, len(_DST_CONTRIBS[d]), tuple(bias_rows), d),
        grid=(-(-n // _B),),
        in_specs=in_specs,
        out_specs=out_specs,
        out_shape=out_shape,
    )(*args)


def _final_c(n, l, d, aggs3, deg, bc, bias_rows, ln_g, ln_b, Wcls, bcls):
    in_specs, args = _c_specs_args(l, d, aggs3, deg, bc, ln_g, ln_b)
    in_specs += [_brow(d, 1, _D_HID, _D_OUT), _b0(3, _D_OUT)]
    args += [Wcls, bcls]
    return pl.pallas_call(
        functools.partial(_final_c_body, len(_DST_CONTRIBS[d]), tuple(bias_rows), d),
        grid=(-(-n // _B),),
        in_specs=in_specs,
        out_specs=pl.BlockSpec((_B, _D_OUT), lambda i: (i, 0)),
        out_shape=jax.ShapeDtypeStruct((n, _D_OUT), jnp.float32),
    )(*args)


def kernel(assmpt_feat, rule_feat, non_assmpt_feat, W_emb, b_emb, Wc, bc,
           ln_g, ln_b, Wcls, bcls, edges_src, edges_dst):
    feats = (assmpt_feat, rule_feat, non_assmpt_feat)
    npad = _EPAD - _E

    gidx_l, degsrc_l, sdst_l = [], [], []
    for e, (s, d) in enumerate(_ETYPES):
        n_s, n_d = _N_TYPES[s], _N_TYPES[d]
        src_e, dst_e = edges_src[e], edges_dst[e]
        gidx_l.append(jnp.concatenate(
            [src_e, jnp.zeros((npad,), jnp.int32)]))
        degsrc_l.append(jnp.concatenate(
            [src_e, jnp.full((npad,), n_s, jnp.int32)]))
        sdst_l.append(jnp.concatenate(
            [dst_e, jnp.full((npad,), n_d, jnp.int32)]))
    gidx = jnp.stack(gidx_l).reshape(9, _NS, _RPT * _CH)
    degsrc = jnp.stack(degsrc_l).reshape(9, _NS, _RPT * _CH)
    sdst = jnp.stack(sdst_l).reshape(9, _NS, _RPT * _CH)
    ones8 = jnp.ones((_RCH * _CH, 8), jnp.float32)
    max_q8 = max(_deg_rows(n) for n in _N_TYPES) // _NS
    zrows8 = jnp.zeros((max_q8, 8), jnp.float32)
    zrows = jnp.zeros((_ZCH, _D_HID), jnp.float32)

    deg = _sc_degrees(degsrc, sdst, ones8, zrows8)

    ztabs = [_embed_a(feats[s], s, W_emb, b_emb, Wc, deg)
             for s in range(3)]

    out = [None, None, None]
    for l in range(_N_LAYERS):
        aggs3 = [_sc_feat_scatter(s, ztabs[s], gidx, sdst, zrows)
                 for s in range(3)]
        last = l == _N_LAYERS - 1
        nxt = []
        for d in range(3):
            n_d = _N_TYPES[d]
            if last:
                out[d] = _final_c(
                    n_d, l, d, aggs3, deg, bc, _DST_GROUPS[d],
                    ln_g, ln_b, Wcls, bcls)
            else:
                nxt.append(_fused_ca(
                    n_d, l, d, aggs3, deg, bc, _DST_GROUPS[d],
                    ln_g, ln_b, Wc))
        ztabs = nxt

    return out[0], out[1], out[2]
